# baseline (speedup 1.0000x reference)
import jax
from jax.experimental import pallas as pl
from jax.experimental.pallas import tpu as pltpu


def _chunk_plan(seq_len):
    if seq_len % 4 == 0 and seq_len >= 8:
        q = seq_len // 4
        return [(0, q), (q, 3 * q)]
    return [(0, seq_len)]


def _make_body(chunks):
    n = len(chunks)

    def body(pe_ref, out_ref, buf, in_sems, out_sems):
        def cp_in(i):
            off, ln = chunks[i]
            return pltpu.make_async_copy(
                pe_ref.at[:, pl.ds(off, ln), :],
                buf.at[:, pl.ds(off, ln), :],
                in_sems.at[i],
            )

        def cp_out(i):
            off, ln = chunks[i]
            return pltpu.make_async_copy(
                buf.at[:, pl.ds(off, ln), :],
                out_ref.at[:, pl.ds(off, ln), :],
                out_sems.at[i],
            )

        for i in range(n):
            cp_in(i).start()
        for i in range(n):
            cp_in(i).wait()
            cp_out(i).start()
        for i in range(n):
            cp_out(i).wait()

    return body


def kernel(x, pe):
    seq_len = x.shape[1]
    d_model = pe.shape[2]
    chunks = _chunk_plan(seq_len)
    n = len(chunks)
    out_shape = jax.ShapeDtypeStruct((1, seq_len, d_model), pe.dtype)
    return pl.pallas_call(
        _make_body(chunks),
        out_shape=out_shape,
        in_specs=[pl.BlockSpec(memory_space=pl.ANY)],
        out_specs=pl.BlockSpec(memory_space=pl.ANY),
        scratch_shapes=[
            pltpu.VMEM((1, seq_len, d_model), pe.dtype),
            pltpu.SemaphoreType.DMA((n,)),
            pltpu.SemaphoreType.DMA((n,)),
        ],
    )(pe)


# speedup vs baseline: 1.1360x; 1.1360x over previous
import jax
from jax.experimental import pallas as pl
from jax.experimental.pallas import tpu as pltpu


def _chunk_plan(seq_len):
    if seq_len % 2 == 0 and seq_len >= 4:
        h = seq_len // 2
        return [(0, h), (h, h)]
    return [(0, seq_len)]


def _make_body(chunks):
    n = len(chunks)

    def body(pe_ref, out_ref, buf, in_sems, out_sems):
        def cp_in(i):
            off, ln = chunks[i]
            return pltpu.make_async_copy(
                pe_ref.at[:, pl.ds(off, ln), :],
                buf.at[:, pl.ds(off, ln), :],
                in_sems.at[i],
            )

        def cp_out(i):
            off, ln = chunks[i]
            return pltpu.make_async_copy(
                buf.at[:, pl.ds(off, ln), :],
                out_ref.at[:, pl.ds(off, ln), :],
                out_sems.at[i],
            )

        for i in range(n):
            cp_in(i).start()
        for i in range(n):
            cp_in(i).wait()
            cp_out(i).start()
        for i in range(n):
            cp_out(i).wait()

    return body


def kernel(x, pe):
    seq_len = x.shape[1]
    d_model = pe.shape[2]
    chunks = _chunk_plan(seq_len)
    n = len(chunks)
    out_shape = jax.ShapeDtypeStruct((1, seq_len, d_model), pe.dtype)
    return pl.pallas_call(
        _make_body(chunks),
        out_shape=out_shape,
        in_specs=[pl.BlockSpec(memory_space=pl.ANY)],
        out_specs=pl.BlockSpec(memory_space=pl.ANY),
        scratch_shapes=[
            pltpu.VMEM((1, seq_len, d_model), pe.dtype),
            pltpu.SemaphoreType.DMA((n,)),
            pltpu.SemaphoreType.DMA((n,)),
        ],
    )(pe)
